# Initial kernel scaffold; baseline (speedup 1.0000x reference)
import jax
import jax.numpy as jnp
from jax.experimental import pallas as pl


def kernel(idx, tok_emb, pos_emb):
    raise NotImplementedError("write your pallas kernel here")


# speedup vs baseline: 1.1461x; 1.1461x over previous
import functools

import jax
import jax.numpy as jnp
from jax import lax
from jax.experimental import pallas as pl
from jax.experimental.pallas import tpu as pltpu
from jax.experimental.pallas import tpu_sc as plsc

NC = 2
NS = 16
NW = NC * NS
LANES = 16

CHUNK_ROWS = 8


def _sc_embed(idx_flat, tok_flat, posT_flat, B, T, V, D):
    rows_per_w = B // NW
    n_chunks = rows_per_w // CHUNK_ROWS
    chunk_tok = CHUNK_ROWS * T
    groups = chunk_tok // LANES
    gpr = T // LANES
    mesh = plsc.VectorSubcoreMesh(
        core_axis_name="c", subcore_axis_name="s", num_cores=NC, num_subcores=NS
    )

    @functools.partial(
        pl.kernel,
        out_type=jax.ShapeDtypeStruct((B * T * D,), jnp.float32),
        mesh=mesh,
        compiler_params=pltpu.CompilerParams(needs_layout_passes=False),
        scratch_types=[
            pltpu.VMEM((V * D,), jnp.float32),
            pltpu.VMEM((T * D,), jnp.float32),
            pltpu.VMEM((chunk_tok,), jnp.int32),
            pltpu.VMEM((chunk_tok * D,), jnp.float32),
        ],
    )
    def k(idx_hbm, tok_hbm, posT_hbm, out_hbm, tok_v, pos_v, idx_v, out_v):
        wid = lax.axis_index("s") * NC + lax.axis_index("c")
        pltpu.sync_copy(tok_hbm, tok_v)
        pltpu.sync_copy(posT_hbm, pos_v)
        iota = lax.iota(jnp.int32, LANES)
        iotaD = iota * D
        tok0_w = wid * rows_per_w * T

        def chunk_body(c, carry):
            tok0 = tok0_w + c * chunk_tok
            pltpu.sync_copy(idx_hbm.at[pl.ds(tok0, chunk_tok)], idx_v)

            def g_body(g, carry2):
                idx_vec = idx_v[pl.ds(g * LANES, LANES)]
                rowbase = idx_vec * D
                toff = jnp.bitwise_and(g, gpr - 1) * LANES
                outbase = jnp.full((LANES,), g * (LANES * D), jnp.int32) + iotaD
                for d in range(D):
                    gath = plsc.load_gather(tok_v, [rowbase + d])
                    pos = pos_v[pl.ds(d * T + toff, LANES)]
                    plsc.store_scatter(out_v, [outbase + d], gath + pos)
                return carry2

            lax.fori_loop(0, groups, g_body, 0)
            pltpu.sync_copy(out_v, out_hbm.at[pl.ds(tok0 * D, chunk_tok * D)])
            return carry

        lax.fori_loop(0, n_chunks, chunk_body, 0)

    return k(idx_flat, tok_flat, posT_flat)


def kernel(idx, tok_emb, pos_emb):
    B, T = idx.shape
    V, D = tok_emb.shape
    out_flat = _sc_embed(
        idx.reshape(-1),
        tok_emb.reshape(-1),
        jnp.transpose(pos_emb).reshape(-1),
        B, T, V, D,
    )
    return out_flat.reshape(B, T, D)


# speedup vs baseline: 1.6964x; 1.4802x over previous
import functools

import jax
import jax.numpy as jnp
from jax import lax
from jax.experimental import pallas as pl
from jax.experimental.pallas import tpu as pltpu
from jax.experimental.pallas import tpu_sc as plsc

NC = 2
NS = 16
NW = NC * NS
LANES = 16

CHUNK_ROWS = 8


def _sc_embed(idx_flat, tok_flat, posT_flat, B, T, V, D):
    rows_per_w = B // NW
    n_chunks = rows_per_w // CHUNK_ROWS
    chunk_tok = CHUNK_ROWS * T
    groups = chunk_tok // LANES
    gpr = T // LANES
    mesh = plsc.VectorSubcoreMesh(
        core_axis_name="c", subcore_axis_name="s", num_cores=NC, num_subcores=NS
    )

    @functools.partial(
        pl.kernel,
        out_type=jax.ShapeDtypeStruct((B * T * D,), jnp.float32),
        mesh=mesh,
        compiler_params=pltpu.CompilerParams(needs_layout_passes=False),
        scratch_types=[
            pltpu.VMEM((V * D,), jnp.float32),
            pltpu.VMEM((T * D,), jnp.float32),
            pltpu.VMEM((chunk_tok,), jnp.int32),
            pltpu.VMEM((chunk_tok * D,), jnp.float32),
        ],
    )
    def k(idx_hbm, tok_hbm, posT_hbm, out_hbm, tok_v, pos_v, idx_v, out_v):
        wid = lax.axis_index("s") * NC + lax.axis_index("c")
        pltpu.sync_copy(tok_hbm, tok_v)
        pltpu.sync_copy(posT_hbm, pos_v)
        iota = lax.iota(jnp.int32, LANES)
        iotaD = iota * D
        tok0_w = wid * rows_per_w * T

        def chunk_body(c, carry):
            tok0 = tok0_w + c * chunk_tok
            pltpu.sync_copy(idx_hbm.at[pl.ds(tok0, chunk_tok)], idx_v)

            @plsc.parallel_loop(0, groups)
            def g_body(g):
                idx_vec = idx_v[pl.ds(g * LANES, LANES)]
                rowbase = idx_vec * D
                toff = jnp.bitwise_and(g, gpr - 1) * LANES
                outbase = jnp.full((LANES,), g * (LANES * D), jnp.int32) + iotaD
                for d in range(D):
                    gath = plsc.load_gather(tok_v, [rowbase + d])
                    pos = pos_v[pl.ds(d * T + toff, LANES)]
                    plsc.store_scatter(out_v, [outbase + d], gath + pos)
            pltpu.sync_copy(out_v, out_hbm.at[pl.ds(tok0 * D, chunk_tok * D)])
            return carry

        lax.fori_loop(0, n_chunks, chunk_body, 0)

    return k(idx_flat, tok_flat, posT_flat)


def kernel(idx, tok_emb, pos_emb):
    B, T = idx.shape
    V, D = tok_emb.shape
    out_flat = _sc_embed(
        idx.reshape(-1),
        tok_emb.reshape(-1),
        jnp.transpose(pos_emb).reshape(-1),
        B, T, V, D,
    )
    return out_flat.reshape(B, T, D)


# speedup vs baseline: 4.1104x; 2.4230x over previous
import functools

import jax
import jax.numpy as jnp
from jax import lax
from jax.experimental import pallas as pl
from jax.experimental.pallas import tpu as pltpu
from jax.experimental.pallas import tpu_sc as plsc

NC = 2
NS = 16
NW = NC * NS
LANES = 16

CHUNK_TOK = 512
GSUB = 128


def _sc_embed(idx_flat, tok_flat, pos_flat, B, T, V, D):
    ntok = B * T
    tok_per_w = ntok // NW
    n_chunks = tok_per_w // CHUNK_TOK
    ngath = CHUNK_TOK // GSUB
    groups16 = CHUNK_TOK // LANES
    tv = T * V
    rows_per_tile = tv // NS
    t_per_tile = rows_per_tile // V
    mesh = plsc.VectorSubcoreMesh(
        core_axis_name="c", subcore_axis_name="s", num_cores=NC, num_subcores=NS
    )

    @functools.partial(
        pl.kernel,
        out_type=(
            jax.ShapeDtypeStruct((ntok, D), jnp.float32),
            jax.ShapeDtypeStruct((NC * tv, D), jnp.float32),
        ),
        mesh=mesh,
        compiler_params=pltpu.CompilerParams(
            needs_layout_passes=False, use_tc_tiling_on_sc=False
        ),
        scratch_types=[
            pltpu.VMEM((V * D,), jnp.float32),
            pltpu.VMEM((T * D,), jnp.float32),
            pltpu.VMEM((rows_per_tile, D), jnp.float32),
            pltpu.VMEM((CHUNK_TOK,), jnp.int32),
            pltpu.VMEM((CHUNK_TOK,), jnp.int32),
            pltpu.VMEM((CHUNK_TOK,), jnp.int32),
            pltpu.VMEM((CHUNK_TOK, D), jnp.float32),
            pltpu.SemaphoreType.DMA,
        ],
    )
    def k(idx_hbm, tok_hbm, pos_hbm, out_hbm, fused_hbm,
          tok_v, pos_v, build_v, idx_v, tbase_v, fidx_v, rows_v, sem):
        c = lax.axis_index("c")
        s = lax.axis_index("s")
        wid = s * NC + c
        pltpu.sync_copy(tok_hbm, tok_v)
        pltpu.sync_copy(pos_hbm, pos_v)
        iota = lax.iota(jnp.int32, LANES)

        for dd in range(D // LANES):
            pos_chunks = [
                pos_v[pl.ds((s * t_per_tile + tt) * D + dd * LANES, LANES)]
                for tt in range(t_per_tile)
            ]
            for v in range(V):
                tokc = tok_v[pl.ds(v * D + dd * LANES, LANES)]
                for tt in range(t_per_tile):
                    build_v[tt * V + v, pl.ds(dd * LANES, LANES)] = (
                        tokc + pos_chunks[tt]
                    )
        pltpu.sync_copy(
            build_v,
            fused_hbm.at[pl.ds(c * tv + s * rows_per_tile, rows_per_tile)],
        )
        plsc.subcore_barrier()

        cbase = c * tv
        for g in range(groups16):
            toff = (g % (T // LANES)) * LANES
            tbase_v[pl.ds(g * LANES, LANES)] = (iota + toff) * V + cbase

        tok0_w = wid * tok_per_w

        def chunk_body(ch, carry):
            tok0 = tok0_w + ch * CHUNK_TOK
            pltpu.sync_copy(idx_hbm.at[pl.ds(tok0, CHUNK_TOK)], idx_v)
            for g in range(groups16):
                fidx_v[pl.ds(g * LANES, LANES)] = (
                    idx_v[pl.ds(g * LANES, LANES)]
                    + tbase_v[pl.ds(g * LANES, LANES)]
                )
            copies = [
                pltpu.async_copy(
                    fused_hbm.at[fidx_v.at[pl.ds(j * GSUB, GSUB)]],
                    rows_v.at[pl.ds(j * GSUB, GSUB)],
                    sem,
                )
                for j in range(ngath)
            ]
            for cp in copies:
                cp.wait()
            pltpu.sync_copy(rows_v, out_hbm.at[pl.ds(tok0, CHUNK_TOK)])
            return carry

        lax.fori_loop(0, n_chunks, chunk_body, 0)

    out2d, _ = k(idx_flat, tok_flat, pos_flat)
    return out2d


def kernel(idx, tok_emb, pos_emb):
    B, T = idx.shape
    V, D = tok_emb.shape
    out2d = _sc_embed(
        idx.reshape(-1),
        tok_emb.reshape(-1),
        pos_emb.reshape(-1),
        B, T, V, D,
    )
    return out2d.reshape(B, T, D)
